# baseline (speedup 1.0000x reference)
import functools

import jax
import jax.numpy as jnp
from jax import lax
from jax.experimental import pallas as pl
from jax.experimental.pallas import tpu as pltpu
from jax.experimental.pallas import tpu_sc as plsc

NC, NS = 2, 16
NW = NC * NS
D = 32
BB = 128


@functools.cache
def _make_gather(NB, S, V):
    n_jb = NB // BB
    j_per_w = n_jb // NW
    n_items = j_per_w * S
    assert NB % (BB * NW) == 0 and n_items % 2 == 0

    RG = 8
    RS = 4
    assert n_items % RG == 0

    mesh = plsc.VectorSubcoreMesh(core_axis_name="c", subcore_axis_name="s")

    scratch = [
        pltpu.VMEM((BB, S), jnp.int32),
        pltpu.VMEM((RG, BB), jnp.int32),
    ]
    scratch += [pltpu.VMEM((BB, D), jnp.float32) for _ in range(RG)]
    scratch += [pltpu.VMEM((D, BB), jnp.float32) for _ in range(RS)]
    scratch += [pltpu.SemaphoreType.DMA for _ in range(RG + RS)]

    @functools.partial(
        pl.kernel,
        out_type=jax.ShapeDtypeStruct((S, D // 8, NB // BB, 8, BB), jnp.float32),
        mesh=mesh,
        scratch_types=scratch,
        compiler_params=pltpu.CompilerParams(
            use_tc_tiling_on_sc=False, needs_layout_passes=False
        ),
    )
    def gather_kernel(idx_hbm, table_hbm, out_hbm, idxb, idxt, *rest):
        rows = rest[:RG]
        cmaj = rest[RG:RG + RS]
        sem_g = rest[RG + RS:2 * RG + RS]
        sem_s = rest[2 * RG + RS:]
        wid = lax.axis_index("s") * NC + lax.axis_index("c")
        iota = lax.iota(jnp.int32, 16)
        h16 = [iota + 16 * h for h in range(8)]

        def item_js(it):
            return wid + (it // S) * NW, it % S

        def fire_item(it, slot):
            j, s = item_js(it)

            @pl.when(s == 0)
            def _load_idx():
                pltpu.sync_copy(
                    idx_hbm.at[pl.ds(pl.multiple_of(j * BB, BB), BB), :], idxb
                )

            for h in range(8):
                g = plsc.load_gather(idxb, [h16[h], jnp.full((16,), s, jnp.int32)])
                idxt[slot, pl.ds(16 * h, 16)] = g
            pltpu.async_copy(
                table_hbm.at[idxt.at[slot]], rows[slot], sem_g[slot]
            )

        def wait_gather(slot):
            pltpu.make_async_copy(
                table_hbm.at[idxt.at[slot]], rows[slot], sem_g[slot]
            ).wait()

        def wait_stores(slot):
            for i in range(D // 8):
                pltpu.make_async_copy(
                    cmaj[slot].at[pl.ds(8 * i, 8), :],
                    out_hbm.at[0, i, 0, :, :],
                    sem_s[slot],
                ).wait()

        def drain_item(it, g, cs):
            j, s = item_js(it)
            wait_gather(g)

            @plsc.parallel_loop(0, D, unroll=8)
            def _tr(c):
                c16 = jnp.full((16,), c, jnp.int32)
                gs = [
                    plsc.load_gather(rows[g], [h16[h], c16]) for h in range(8)
                ]
                for h in range(8):
                    cmaj[cs][c, pl.ds(16 * h, 16)] = gs[h]
            for i in range(D // 8):
                pltpu.async_copy(
                    cmaj[cs].at[pl.ds(8 * i, 8), :],
                    out_hbm.at[s, i, j, :, :],
                    sem_s[cs],
                )

        for g in range(RG):
            fire_item(g, g)

        @pl.loop(0, n_items // RG)
        def ring(itr):
            for g in range(RG):
                it = itr * RG + g
                cs = g % RS

                @pl.when(it >= RS)
                def _drain_store(cs=cs):
                    wait_stores(cs)

                drain_item(it, g, cs)

                @pl.when(it + RG < n_items)
                def _refill(it=it, g=g):
                    fire_item(it + RG, g)

        for cs in range(RS):
            wait_stores(cs)

    return gather_kernel


def kernel(token_ids, E):
    V, d = E.shape
    NB, S = token_ids.shape
    assert d == D
    table_wide = lax.optimization_barrier(E.reshape(V * D // 128, 128))
    table = table_wide.reshape(V, D)
    z5 = _make_gather(NB, S, V)(token_ids.astype(jnp.int32), table)
    return z5.transpose(2, 4, 0, 1, 3).reshape(NB, S, D)


# speedup vs baseline: 1.1402x; 1.1402x over previous
import functools

import jax
import jax.numpy as jnp
from jax import lax
from jax.experimental import pallas as pl
from jax.experimental.pallas import tpu as pltpu
from jax.experimental.pallas import tpu_sc as plsc

NC, NS = 2, 16
NW = NC * NS
D = 32
BB = 128


@functools.cache
def _make_gather(NB, S, V):
    n_jb = NB // BB
    j_per_w = n_jb // NW
    n_items = j_per_w * S
    assert NB % (BB * NW) == 0 and n_items % 2 == 0

    RG = 8
    RS = 4
    assert n_items % RG == 0

    mesh = plsc.VectorSubcoreMesh(core_axis_name="c", subcore_axis_name="s")

    scratch = [
        pltpu.VMEM((BB, S), jnp.int32),
        pltpu.VMEM((RG, BB), jnp.int32),
    ]
    scratch += [pltpu.VMEM((BB, D), jnp.float32) for _ in range(RG)]
    scratch += [pltpu.VMEM((D, BB), jnp.float32) for _ in range(RS)]
    scratch += [pltpu.SemaphoreType.DMA for _ in range(RG + RS)]

    @functools.partial(
        pl.kernel,
        out_type=jax.ShapeDtypeStruct((S, D // 8, NB // BB, 8, BB), jnp.float32),
        mesh=mesh,
        scratch_types=scratch,
        compiler_params=pltpu.CompilerParams(
            use_tc_tiling_on_sc=False, needs_layout_passes=False
        ),
    )
    def gather_kernel(idx_hbm, table_hbm, out_hbm, idxb, idxt, *rest):
        rows = rest[:RG]
        cmaj = rest[RG:RG + RS]
        sem_g = rest[RG + RS:2 * RG + RS]
        sem_s = rest[2 * RG + RS:]
        wid = lax.axis_index("s") * NC + lax.axis_index("c")
        iota = lax.iota(jnp.int32, 16)
        h16 = [iota + 16 * h for h in range(8)]

        def item_js(it):
            return wid + (it // S) * NW, it % S

        def fire_item(it, slot):
            j, s = item_js(it)

            @pl.when(s == 0)
            def _load_idx():
                pltpu.sync_copy(
                    idx_hbm.at[pl.ds(pl.multiple_of(j * BB, BB), BB), :], idxb
                )

            s16 = jnp.full((16,), s, jnp.int32)

            @plsc.parallel_loop(0, 8, unroll=8)
            def _tri(h):
                hv = iota + h * 16
                g = plsc.load_gather(idxb, [hv, s16])
                idxt[slot, pl.ds(h * 16, 16)] = g
            pltpu.async_copy(
                table_hbm.at[idxt.at[slot]], rows[slot], sem_g[slot]
            )

        def wait_gather(slot):
            pltpu.make_async_copy(
                table_hbm.at[idxt.at[slot]], rows[slot], sem_g[slot]
            ).wait()

        def wait_stores(slot):
            for i in range(D // 8):
                pltpu.make_async_copy(
                    cmaj[slot].at[pl.ds(8 * i, 8), :],
                    out_hbm.at[0, i, 0, :, :],
                    sem_s[slot],
                ).wait()

        def drain_item(it, g, cs):
            j, s = item_js(it)
            wait_gather(g)

            @plsc.parallel_loop(0, D, unroll=16)
            def _tr(c):
                c16 = jnp.full((16,), c, jnp.int32)
                for h in range(8):
                    gv = plsc.load_gather(rows[g], [h16[h], c16])
                    cmaj[cs][c, pl.ds(16 * h, 16)] = gv
            for i in range(D // 8):
                pltpu.async_copy(
                    cmaj[cs].at[pl.ds(8 * i, 8), :],
                    out_hbm.at[s, i, j, :, :],
                    sem_s[cs],
                )

        for g in range(RG):
            fire_item(g, g)

        @pl.loop(0, n_items // RG)
        def ring(itr):
            for g in range(RG):
                it = itr * RG + g
                cs = g % RS

                @pl.when(it >= RS)
                def _drain_store(cs=cs):
                    wait_stores(cs)

                drain_item(it, g, cs)

                @pl.when(it + RG < n_items)
                def _refill(it=it, g=g):
                    fire_item(it + RG, g)

        for cs in range(RS):
            wait_stores(cs)

    return gather_kernel


def kernel(token_ids, E):
    V, d = E.shape
    NB, S = token_ids.shape
    assert d == D
    table_wide = lax.optimization_barrier(E.reshape(V * D // 128, 128))
    table = table_wide.reshape(V, D)
    z5 = _make_gather(NB, S, V)(token_ids.astype(jnp.int32), table)
    return z5.transpose(2, 4, 0, 1, 3).reshape(NB, S, D)
